# Initial kernel scaffold; baseline (speedup 1.0000x reference)
import jax
import jax.numpy as jnp
from jax.experimental import pallas as pl


def kernel(spikes, weight, bias):
    raise NotImplementedError("write your pallas kernel here")


# speedup vs baseline: 2.3207x; 2.3207x over previous
import functools

import jax
import jax.numpy as jnp
from jax.experimental import pallas as pl

NUM_VARS = 128
K = 5


def _conv_body(x_ref, w_ref, b_ref, out_ref):
    x = x_ref[0]
    t = x.shape[0]
    xp = jnp.pad(x, ((K - 2, 1), (0, 0)))
    acc = jnp.broadcast_to(b_ref[0][None, :], (t, NUM_VARS)).astype(jnp.float32)
    for k in range(K):
        wk = w_ref[k]
        if k == K - 1:
            row = jax.lax.broadcasted_iota(jnp.int32, (NUM_VARS, NUM_VARS), 0)
            col = jax.lax.broadcasted_iota(jnp.int32, (NUM_VARS, NUM_VARS), 1)
            wk = jnp.where(row == col, 0.0, wk)
        acc = acc + jax.lax.dot_general(
            xp[k:k + t], wk,
            dimension_numbers=(((1,), (1,)), ((), ())),
            preferred_element_type=jnp.float32)
    out_ref[0] = acc


@functools.partial(jax.jit, static_argnames=())
def kernel(spikes, weight, bias):
    b, t, n, _ = spikes.shape
    x = spikes[..., 0]
    w = jnp.transpose(weight, (2, 0, 1))
    bias2 = bias[None, :]
    out = pl.pallas_call(
        _conv_body,
        grid=(b,),
        in_specs=[
            pl.BlockSpec((1, t, n), lambda i: (i, 0, 0)),
            pl.BlockSpec((K, n, n), lambda i: (0, 0, 0)),
            pl.BlockSpec((1, n), lambda i: (0, 0)),
        ],
        out_specs=pl.BlockSpec((1, t, n), lambda i: (i, 0, 0)),
        out_shape=jax.ShapeDtypeStruct((b, t, n), jnp.float32),
    )(x, w, bias2)
    return out[:, :t - 1, :, None]


# speedup vs baseline: 2.3218x; 1.0005x over previous
import functools

import jax
import jax.numpy as jnp
from jax.experimental import pallas as pl

NUM_VARS = 128
K = 5


def _conv_body(x_ref, w_ref, b_ref, out_ref):
    x = x_ref[0]
    t = x.shape[0]
    xp = jnp.pad(x, ((K - 2, 1), (0, 0))).astype(jnp.bfloat16)
    acc = jnp.broadcast_to(b_ref[0][None, :], (t, NUM_VARS)).astype(jnp.float32)
    for k in range(K):
        wk = w_ref[k].astype(jnp.bfloat16)
        if k == K - 1:
            row = jax.lax.broadcasted_iota(jnp.int32, (NUM_VARS, NUM_VARS), 0)
            col = jax.lax.broadcasted_iota(jnp.int32, (NUM_VARS, NUM_VARS), 1)
            wk = jnp.where(row == col, 0.0, wk)
        acc = acc + jax.lax.dot_general(
            xp[k:k + t], wk,
            dimension_numbers=(((1,), (1,)), ((), ())),
            preferred_element_type=jnp.float32)
    out_ref[0] = acc


@functools.partial(jax.jit, static_argnames=())
def kernel(spikes, weight, bias):
    b, t, n, _ = spikes.shape
    x = spikes[..., 0]
    w = jnp.transpose(weight, (2, 0, 1))
    bias2 = bias[None, :]
    out = pl.pallas_call(
        _conv_body,
        grid=(b,),
        in_specs=[
            pl.BlockSpec((1, t, n), lambda i: (i, 0, 0)),
            pl.BlockSpec((K, n, n), lambda i: (0, 0, 0)),
            pl.BlockSpec((1, n), lambda i: (0, 0)),
        ],
        out_specs=pl.BlockSpec((1, t, n), lambda i: (i, 0, 0)),
        out_shape=jax.ShapeDtypeStruct((b, t, n), jnp.float32),
    )(x, w, bias2)
    return out[:, :t - 1, :, None]


# speedup vs baseline: 2.9178x; 1.2567x over previous
import functools

import jax
import jax.numpy as jnp
from jax.experimental import pallas as pl

NUM_VARS = 128
K = 5


def _conv_body(x_ref, w_ref, b_ref, out_ref):
    x = x_ref[0]
    t = x.shape[0]
    xp = jnp.pad(x, ((K - 2, 1), (0, 0))).astype(jnp.bfloat16)
    acc = jnp.broadcast_to(b_ref[0][None, :], (t, NUM_VARS)).astype(jnp.float32)
    for k in range(K):
        wk = w_ref[k].astype(jnp.bfloat16)
        if k == K - 1:
            row = jax.lax.broadcasted_iota(jnp.int32, (NUM_VARS, NUM_VARS), 0)
            col = jax.lax.broadcasted_iota(jnp.int32, (NUM_VARS, NUM_VARS), 1)
            wk = jnp.where(row == col, 0.0, wk)
        acc = acc + jax.lax.dot_general(
            xp[k:k + t], wk,
            dimension_numbers=(((1,), (1,)), ((), ())),
            preferred_element_type=jnp.float32)
    out_ref[0] = acc[:t - 1]


@functools.partial(jax.jit, static_argnames=())
def kernel(spikes, weight, bias):
    b, t, n, _ = spikes.shape
    x = jnp.reshape(spikes, (b, t, n))
    w = jnp.transpose(weight, (2, 0, 1))
    bias2 = bias[None, :]
    out = pl.pallas_call(
        _conv_body,
        grid=(b,),
        in_specs=[
            pl.BlockSpec((1, t, n), lambda i: (i, 0, 0)),
            pl.BlockSpec((K, n, n), lambda i: (0, 0, 0)),
            pl.BlockSpec((1, n), lambda i: (0, 0)),
        ],
        out_specs=pl.BlockSpec((1, t - 1, n), lambda i: (i, 0, 0)),
        out_shape=jax.ShapeDtypeStruct((b, t - 1, n), jnp.float32),
    )(x, w, bias2)
    return jnp.reshape(out, (b, t - 1, n, 1))


# speedup vs baseline: 5.3556x; 1.8355x over previous
import functools

import jax
import jax.numpy as jnp
from jax.experimental import pallas as pl

NUM_VARS = 128
K = 5


def _conv_body(x_ref, w_ref, b_ref, out_ref):
    x = x_ref[0]
    t = x.shape[0]
    xp = jnp.pad(x, ((K - 2, 1), (0, 0))).astype(jnp.bfloat16)
    acc = jnp.broadcast_to(b_ref[0][None, :], (t, NUM_VARS)).astype(jnp.float32)
    for k in range(K):
        wk = w_ref[k].astype(jnp.bfloat16)
        if k == K - 1:
            row = jax.lax.broadcasted_iota(jnp.int32, (NUM_VARS, NUM_VARS), 0)
            col = jax.lax.broadcasted_iota(jnp.int32, (NUM_VARS, NUM_VARS), 1)
            wk = jnp.where(row == col, 0.0, wk)
        acc = acc + jax.lax.dot_general(
            xp[k:k + t], wk,
            dimension_numbers=(((1,), (1,)), ((), ())),
            preferred_element_type=jnp.float32)
    out_ref[0, :, 0, :] = acc[:t - 1]


@functools.partial(jax.jit, static_argnames=())
def kernel(spikes, weight, bias):
    b, t, n, _ = spikes.shape
    x = jnp.reshape(spikes, (b, t, n))
    w = jnp.transpose(weight, (2, 0, 1))
    bias2 = bias[None, :]
    out = pl.pallas_call(
        _conv_body,
        grid=(b,),
        in_specs=[
            pl.BlockSpec((1, t, n), lambda i: (i, 0, 0)),
            pl.BlockSpec((K, n, n), lambda i: (0, 0, 0)),
            pl.BlockSpec((1, n), lambda i: (0, 0)),
        ],
        out_specs=pl.BlockSpec((1, t - 1, 1, n), lambda i: (i, 0, 0, 0)),
        out_shape=jax.ShapeDtypeStruct((b, t - 1, 1, n), jnp.float32),
    )(x, w, bias2)
    return jnp.reshape(out, (b, t - 1, n, 1))


# speedup vs baseline: 7.4155x; 1.3846x over previous
import functools

import jax
import jax.numpy as jnp
from jax.experimental import pallas as pl
from jax.experimental.pallas import tpu as pltpu

NUM_VARS = 128
K = 5


def _conv_body(x_ref, w_ref, b_ref, out_hbm, xs_ref, sem):
    i = pl.program_id(0)
    nb = pl.num_programs(0)
    slot = jax.lax.rem(i, 2)
    x = x_ref[0]
    t = x.shape[0]

    @pl.when(i >= 2)
    def _():
        pltpu.make_async_copy(
            xs_ref.at[slot, :t - 1],
            out_hbm.at[i - 2, :, 0, :],
            sem.at[slot]).wait()

    xp = jnp.pad(x, ((K - 2, 1), (0, 0))).astype(jnp.bfloat16)
    acc = jnp.broadcast_to(b_ref[0][None, :], (t, NUM_VARS)).astype(jnp.float32)
    for k in range(K):
        wk = w_ref[k].astype(jnp.bfloat16)
        if k == K - 1:
            row = jax.lax.broadcasted_iota(jnp.int32, (NUM_VARS, NUM_VARS), 0)
            col = jax.lax.broadcasted_iota(jnp.int32, (NUM_VARS, NUM_VARS), 1)
            wk = jnp.where(row == col, 0.0, wk)
        acc = acc + jax.lax.dot_general(
            xp[k:k + t], wk,
            dimension_numbers=(((1,), (1,)), ((), ())),
            preferred_element_type=jnp.float32)
    xs_ref[slot] = acc

    pltpu.make_async_copy(
        xs_ref.at[slot, :t - 1],
        out_hbm.at[i, :, 0, :],
        sem.at[slot]).start()

    @pl.when(i == nb - 1)
    def _():
        @pl.when(nb >= 2)
        def _():
            pltpu.make_async_copy(
                xs_ref.at[1 - slot, :t - 1],
                out_hbm.at[i - 1, :, 0, :],
                sem.at[1 - slot]).wait()
        pltpu.make_async_copy(
            xs_ref.at[slot, :t - 1],
            out_hbm.at[i, :, 0, :],
            sem.at[slot]).wait()


@functools.partial(jax.jit, static_argnames=())
def kernel(spikes, weight, bias):
    b, t, n, _ = spikes.shape
    x = jnp.reshape(spikes, (b, t, n))
    w = jnp.transpose(weight, (2, 0, 1))
    bias2 = bias[None, :]
    out = pl.pallas_call(
        _conv_body,
        grid=(b,),
        in_specs=[
            pl.BlockSpec((1, t, n), lambda i: (i, 0, 0)),
            pl.BlockSpec((K, n, n), lambda i: (0, 0, 0)),
            pl.BlockSpec((1, n), lambda i: (0, 0)),
        ],
        out_specs=pl.BlockSpec(memory_space=pl.ANY),
        out_shape=jax.ShapeDtypeStruct((b, t - 1, 1, n), jnp.float32),
        scratch_shapes=[
            pltpu.MemorySpace.VMEM((2, t, n), jnp.float32),
            pltpu.SemaphoreType.DMA((2,)),
        ],
    )(x, w, bias2)
    return jnp.reshape(out, (b, t - 1, n, 1))
